# baseline (speedup 1.0000x reference)
import functools

import jax
import jax.numpy as jnp
from jax import lax
from jax.experimental import pallas as pl
from jax.experimental.pallas import tpu as pltpu
from jax.experimental.pallas import tpu_sc as plsc

N_NODES = 10000
N_EDGES = 320000
D = 128
HID = 64

NC = 2
NS = 16
NW = NC * NS
EPW = N_EDGES // NW
CHUNK = 80
NCHUNKS = EPW // CHUNK

_mesh = plsc.VectorSubcoreMesh(core_axis_name="c", subcore_axis_name="s")


@functools.partial(
    pl.kernel,
    mesh=_mesh,
    out_type=jax.ShapeDtypeStruct((N_EDGES, D), jnp.float32),
    scratch_types=[
        pltpu.VMEM((EPW,), jnp.int32),
        pltpu.VMEM((EPW,), jnp.int32),
        pltpu.VMEM((CHUNK, D), jnp.float32),
        pltpu.VMEM((CHUNK, D), jnp.float32),
        pltpu.SemaphoreType.DMA,
        pltpu.SemaphoreType.DMA,
    ],
)
def _gather_absdiff(x_hbm, src_hbm, dst_hbm, out_hbm,
                    idx_s, idx_d, buf_a, buf_b, sem_a, sem_b):
    wid = lax.axis_index("s") * NC + lax.axis_index("c")
    base0 = wid * EPW
    pltpu.sync_copy(src_hbm.at[pl.ds(base0, EPW)], idx_s)
    pltpu.sync_copy(dst_hbm.at[pl.ds(base0, EPW)], idx_d)

    def chunk_body(i, carry):
        off = i * CHUNK
        cp_a = pltpu.async_copy(x_hbm.at[idx_s.at[pl.ds(off, CHUNK)]], buf_a, sem_a)
        cp_b = pltpu.async_copy(x_hbm.at[idx_d.at[pl.ds(off, CHUNK)]], buf_b, sem_b)
        cp_a.wait()
        cp_b.wait()

        def row_body(r, c2):
            for c in range(D // 16):
                sl = pl.ds(c * 16, 16)
                buf_a[r, sl] = jnp.abs(buf_a[r, sl] - buf_b[r, sl])
            return c2

        lax.fori_loop(0, CHUNK, row_body, 0)
        pltpu.sync_copy(buf_a, out_hbm.at[pl.ds(base0 + off, CHUNK)])
        return carry

    lax.fori_loop(0, NCHUNKS, chunk_body, 0)


BLK = 2560
NB = N_EDGES // BLK


def _mlp_body(d_ref, w1_ref, b1_ref, w2_ref, b2_ref, o_ref):
    h = jnp.dot(d_ref[...], w1_ref[...], preferred_element_type=jnp.float32)
    h = jnp.maximum(h + b1_ref[...], 0.0)
    logits = jnp.dot(h, w2_ref[...], preferred_element_type=jnp.float32)
    o_ref[...] = jax.nn.sigmoid(logits + b2_ref[...])


def _mlp(diff, w1, b1r, w2, b2r):
    return pl.pallas_call(
        _mlp_body,
        grid=(NB,),
        in_specs=[
            pl.BlockSpec((BLK, D), lambda g: (g, 0)),
            pl.BlockSpec((D, HID), lambda g: (0, 0)),
            pl.BlockSpec((1, HID), lambda g: (0, 0)),
            pl.BlockSpec((HID, 1), lambda g: (0, 0)),
            pl.BlockSpec((1, 1), lambda g: (0, 0)),
        ],
        out_specs=pl.BlockSpec((BLK, 1), lambda g: (g, 0)),
        out_shape=jax.ShapeDtypeStruct((N_EDGES, 1), jnp.float32),
    )(diff, w1, b1r, w2, b2r)


def kernel(x, edge_index, W1, b1, W2, b2):
    ei = edge_index
    src = ei[0]
    dst = ei[1]
    diff = _gather_absdiff(x, src, dst)
    vals = _mlp(diff, W1, b1.reshape(1, HID), W2, b2.reshape(1, 1))
    values = vals.reshape(-1)
    return (ei, values)


# speedup vs baseline: 1.2833x; 1.2833x over previous
import functools

import jax
import jax.numpy as jnp
from jax import lax
from jax.experimental import pallas as pl
from jax.experimental.pallas import tpu as pltpu
from jax.experimental.pallas import tpu_sc as plsc

N_NODES = 10000
N_EDGES = 320000
D = 128
HID = 64

NC = 2
NS = 16
NW = NC * NS
EPW = N_EDGES // NW
CHUNK = 80
NCHUNKS = EPW // CHUNK

_mesh = plsc.VectorSubcoreMesh(core_axis_name="c", subcore_axis_name="s")


@functools.partial(
    pl.kernel,
    mesh=_mesh,
    out_type=jax.ShapeDtypeStruct((N_EDGES, D), jnp.float32),
    scratch_types=[
        pltpu.VMEM((EPW,), jnp.int32),
        pltpu.VMEM((EPW,), jnp.int32),
        pltpu.VMEM((CHUNK, D), jnp.float32),
        pltpu.VMEM((CHUNK, D), jnp.float32),
        pltpu.SemaphoreType.DMA,
        pltpu.SemaphoreType.DMA,
    ],
)
def _gather_absdiff(x_hbm, src_hbm, dst_hbm, out_hbm,
                    idx_s, idx_d, buf_a, buf_b, sem_a, sem_b):
    wid = lax.axis_index("s") * NC + lax.axis_index("c")
    base0 = wid * EPW
    pltpu.sync_copy(src_hbm.at[pl.ds(base0, EPW)], idx_s)
    pltpu.sync_copy(dst_hbm.at[pl.ds(base0, EPW)], idx_d)

    def chunk_body(i, carry):
        off = i * CHUNK
        cp_a = pltpu.async_copy(x_hbm.at[idx_s.at[pl.ds(off, CHUNK)]], buf_a, sem_a)
        cp_b = pltpu.async_copy(x_hbm.at[idx_d.at[pl.ds(off, CHUNK)]], buf_b, sem_b)
        cp_a.wait()
        cp_b.wait()

        def row_body(r, c2):
            for c in range(D // 16):
                sl = pl.ds(c * 16, 16)
                buf_a[r, sl] = jnp.abs(buf_a[r, sl] - buf_b[r, sl])
            return c2

        lax.fori_loop(0, CHUNK, row_body, 0)
        pltpu.sync_copy(buf_a, out_hbm.at[pl.ds(base0 + off, CHUNK)])
        return carry

    lax.fori_loop(0, NCHUNKS, chunk_body, 0)


BLK = 6400
NB = N_EDGES // BLK
OROWS = BLK // 128


def _mlp_body(d_ref, w1_ref, b1_ref, w2_ref, b2_ref, o_ref):
    h = jnp.dot(d_ref[...], w1_ref[...], preferred_element_type=jnp.float32)
    h = jnp.maximum(h + b1_ref[...], 0.0)
    logits = jnp.dot(h, w2_ref[...], preferred_element_type=jnp.float32)
    tile = logits.reshape(1, OROWS, 128)
    o_ref[...] = jax.nn.sigmoid(tile + b2_ref[...])


def _mlp(diff, w1, b1r, w2, b2r):
    return pl.pallas_call(
        _mlp_body,
        grid=(NB,),
        in_specs=[
            pl.BlockSpec((BLK, D), lambda g: (g, 0)),
            pl.BlockSpec((D, HID), lambda g: (0, 0)),
            pl.BlockSpec((1, HID), lambda g: (0, 0)),
            pl.BlockSpec((HID, 1), lambda g: (0, 0)),
            pl.BlockSpec((1, 1), lambda g: (0, 0)),
        ],
        out_specs=pl.BlockSpec((1, OROWS, 128), lambda g: (g, 0, 0)),
        out_shape=jax.ShapeDtypeStruct((NB, OROWS, 128), jnp.float32),
    )(diff, w1, b1r, w2, b2r)


def kernel(x, edge_index, W1, b1, W2, b2):
    ei = edge_index
    src = ei[0]
    dst = ei[1]
    diff = _gather_absdiff(x, src, dst)
    vals = _mlp(diff, W1, b1.reshape(1, HID), W2, b2.reshape(1, 1))
    values = vals.reshape(-1)
    return (ei, values)


# speedup vs baseline: 1.5423x; 1.2018x over previous
import functools

import jax
import jax.numpy as jnp
from jax import lax
from jax.experimental import pallas as pl
from jax.experimental.pallas import tpu as pltpu
from jax.experimental.pallas import tpu_sc as plsc

N_NODES = 10000
N_EDGES = 320000
D = 128
HID = 64

NC = 2
NS = 16
NW = NC * NS
EPW = N_EDGES // NW
CHUNK = 80
NCHUNKS = EPW // CHUNK

_mesh = plsc.VectorSubcoreMesh(core_axis_name="c", subcore_axis_name="s")


@functools.partial(
    pl.kernel,
    mesh=_mesh,
    out_type=jax.ShapeDtypeStruct((N_EDGES, D), jnp.float32),
    scratch_types=[
        pltpu.VMEM((EPW,), jnp.int32),
        pltpu.VMEM((EPW,), jnp.int32),
        pltpu.VMEM((CHUNK, D), jnp.float32),
        pltpu.VMEM((CHUNK, D), jnp.float32),
        pltpu.VMEM_SHARED((N_NODES, D), jnp.float32),
        pltpu.SemaphoreType.DMA,
        pltpu.SemaphoreType.DMA,
    ],
)
def _gather_absdiff(x_hbm, src_hbm, dst_hbm, out_hbm,
                    idx_s, idx_d, buf_a, buf_b, x_sh, sem_a, sem_b):
    sid = lax.axis_index("s")
    wid = sid * NC + lax.axis_index("c")
    base0 = wid * EPW

    @pl.when(sid == 0)
    def _():
        pltpu.sync_copy(x_hbm, x_sh)

    plsc.subcore_barrier()

    pltpu.sync_copy(src_hbm.at[pl.ds(base0, EPW)], idx_s)
    pltpu.sync_copy(dst_hbm.at[pl.ds(base0, EPW)], idx_d)

    def chunk_body(i, carry):
        off = i * CHUNK
        cp_a = pltpu.async_copy(x_sh.at[idx_s.at[pl.ds(off, CHUNK)]], buf_a, sem_a)
        cp_b = pltpu.async_copy(x_sh.at[idx_d.at[pl.ds(off, CHUNK)]], buf_b, sem_b)
        cp_a.wait()
        cp_b.wait()

        def row_body(r, c2):
            for c in range(D // 16):
                sl = pl.ds(c * 16, 16)
                buf_a[r, sl] = jnp.abs(buf_a[r, sl] - buf_b[r, sl])
            return c2

        lax.fori_loop(0, CHUNK, row_body, 0)
        pltpu.sync_copy(buf_a, out_hbm.at[pl.ds(base0 + off, CHUNK)])
        return carry

    lax.fori_loop(0, NCHUNKS, chunk_body, 0)


BLK = 6400
NB = N_EDGES // BLK
OROWS = BLK // 128


def _mlp_body(d_ref, w1_ref, b1_ref, w2_ref, b2_ref, o_ref):
    h = jnp.dot(d_ref[...], w1_ref[...], preferred_element_type=jnp.float32)
    h = jnp.maximum(h + b1_ref[...], 0.0)
    logits = jnp.dot(h, w2_ref[...], preferred_element_type=jnp.float32)
    tile = logits.reshape(1, OROWS, 128)
    o_ref[...] = jax.nn.sigmoid(tile + b2_ref[...])


def _mlp(diff, w1, b1r, w2, b2r):
    return pl.pallas_call(
        _mlp_body,
        grid=(NB,),
        in_specs=[
            pl.BlockSpec((BLK, D), lambda g: (g, 0)),
            pl.BlockSpec((D, HID), lambda g: (0, 0)),
            pl.BlockSpec((1, HID), lambda g: (0, 0)),
            pl.BlockSpec((HID, 1), lambda g: (0, 0)),
            pl.BlockSpec((1, 1), lambda g: (0, 0)),
        ],
        out_specs=pl.BlockSpec((1, OROWS, 128), lambda g: (g, 0, 0)),
        out_shape=jax.ShapeDtypeStruct((NB, OROWS, 128), jnp.float32),
    )(diff, w1, b1r, w2, b2r)


def kernel(x, edge_index, W1, b1, W2, b2):
    ei = edge_index
    src = ei[0]
    dst = ei[1]
    diff = _gather_absdiff(x, src, dst)
    vals = _mlp(diff, W1, b1.reshape(1, HID), W2, b2.reshape(1, 1))
    values = vals.reshape(-1)
    return (ei, values)


# speedup vs baseline: 2.1842x; 1.4162x over previous
import functools

import jax
import jax.numpy as jnp
from jax import lax
from jax.experimental import pallas as pl
from jax.experimental.pallas import tpu as pltpu
from jax.experimental.pallas import tpu_sc as plsc

N_NODES = 10000
N_EDGES = 320000
D = 128
HID = 64

NC = 2
NS = 16
NW = NC * NS
EPW = N_EDGES // NW
CHUNK = 40
NCHUNKS = EPW // CHUNK
NPAIR = NCHUNKS // 2

_mesh = plsc.VectorSubcoreMesh(core_axis_name="c", subcore_axis_name="s")


@functools.partial(
    pl.kernel,
    mesh=_mesh,
    out_type=jax.ShapeDtypeStruct((N_EDGES, D), jnp.float32),
    scratch_types=[
        pltpu.VMEM((EPW,), jnp.int32),
        pltpu.VMEM((EPW,), jnp.int32),
        pltpu.VMEM((CHUNK, D), jnp.float32),
        pltpu.VMEM((CHUNK, D), jnp.float32),
        pltpu.VMEM((CHUNK, D), jnp.float32),
        pltpu.VMEM((CHUNK, D), jnp.float32),
        pltpu.VMEM((CHUNK, D), jnp.float32),
        pltpu.VMEM((CHUNK, D), jnp.float32),
        pltpu.VMEM_SHARED((N_NODES, D), jnp.float32),
        pltpu.SemaphoreType.DMA,
        pltpu.SemaphoreType.DMA,
        pltpu.SemaphoreType.DMA,
        pltpu.SemaphoreType.DMA,
        pltpu.SemaphoreType.DMA,
        pltpu.SemaphoreType.DMA,
    ],
)
def _gather_absdiff(x_hbm, src_hbm, dst_hbm, out_hbm,
                    idx_s, idx_d, a0, a1, b0, b1, o0, o1, x_sh,
                    sga0, sga1, sgb0, sgb1, sst0, sst1):
    bufs_a = (a0, a1)
    bufs_b = (b0, b1)
    bufs_o = (o0, o1)
    sems_a = (sga0, sga1)
    sems_b = (sgb0, sgb1)
    sems_o = (sst0, sst1)

    sid = lax.axis_index("s")
    wid = sid * NC + lax.axis_index("c")
    base0 = wid * EPW

    @pl.when(sid == 0)
    def _():
        pltpu.sync_copy(x_hbm, x_sh)

    plsc.subcore_barrier()

    pltpu.sync_copy(src_hbm.at[pl.ds(base0, EPW)], idx_s)
    pltpu.sync_copy(dst_hbm.at[pl.ds(base0, EPW)], idx_d)

    def issue_gathers(chunk, s):
        off = chunk * CHUNK
        pltpu.async_copy(x_sh.at[idx_s.at[pl.ds(off, CHUNK)]], bufs_a[s],
                         sems_a[s])
        pltpu.async_copy(x_sh.at[idx_d.at[pl.ds(off, CHUNK)]], bufs_b[s],
                         sems_b[s])

    def wait_gathers(s):
        pltpu.make_async_copy(out_hbm.at[pl.ds(0, CHUNK)], bufs_a[s],
                              sems_a[s]).wait()
        pltpu.make_async_copy(out_hbm.at[pl.ds(0, CHUNK)], bufs_b[s],
                              sems_b[s]).wait()

    def wait_store(s):
        pltpu.make_async_copy(bufs_o[s], out_hbm.at[pl.ds(0, CHUNK)],
                              sems_o[s]).wait()

    issue_gathers(0, 0)
    issue_gathers(1, 1)

    def pair_body(i, carry):
        for s in (0, 1):
            chunk = i * 2 + s
            wait_gathers(s)

            @pl.when(i > 0)
            def _():
                wait_store(s)

            buf_a, buf_b, buf_o = bufs_a[s], bufs_b[s], bufs_o[s]

            def row_body(r, c2):
                for c in range(D // 16):
                    sl = pl.ds(c * 16, 16)
                    buf_o[r, sl] = jnp.abs(buf_a[r, sl] - buf_b[r, sl])
                return c2

            lax.fori_loop(0, CHUNK, row_body, 0)

            pltpu.async_copy(buf_o, out_hbm.at[pl.ds(base0 + chunk * CHUNK,
                                                     CHUNK)], sems_o[s])

            @pl.when(chunk + 2 < NCHUNKS)
            def _():
                issue_gathers(chunk + 2, s)

        return carry

    lax.fori_loop(0, NPAIR, pair_body, 0)
    wait_store(0)
    wait_store(1)


BLK = 6400
NB = N_EDGES // BLK
OROWS = BLK // 128


def _mlp_body(d_ref, w1_ref, b1_ref, w2_ref, b2_ref, o_ref):
    h = jnp.dot(d_ref[...], w1_ref[...], preferred_element_type=jnp.float32)
    h = jnp.maximum(h + b1_ref[...], 0.0)
    logits = jnp.dot(h, w2_ref[...], preferred_element_type=jnp.float32)
    tile = logits.reshape(1, OROWS, 128)
    o_ref[...] = jax.nn.sigmoid(tile + b2_ref[...])


def _mlp(diff, w1, b1r, w2, b2r):
    return pl.pallas_call(
        _mlp_body,
        grid=(NB,),
        in_specs=[
            pl.BlockSpec((BLK, D), lambda g: (g, 0)),
            pl.BlockSpec((D, HID), lambda g: (0, 0)),
            pl.BlockSpec((1, HID), lambda g: (0, 0)),
            pl.BlockSpec((HID, 1), lambda g: (0, 0)),
            pl.BlockSpec((1, 1), lambda g: (0, 0)),
        ],
        out_specs=pl.BlockSpec((1, OROWS, 128), lambda g: (g, 0, 0)),
        out_shape=jax.ShapeDtypeStruct((NB, OROWS, 128), jnp.float32),
    )(diff, w1, b1r, w2, b2r)


def kernel(x, edge_index, W1, b1, W2, b2):
    ei = edge_index
    src = ei[0]
    dst = ei[1]
    diff = _gather_absdiff(x, src, dst)
    vals = _mlp(diff, W1, b1.reshape(1, HID), W2, b2.reshape(1, 1))
    values = vals.reshape(-1)
    return (ei, values)
